# baseline (speedup 1.0000x reference)
import functools

import jax
import jax.numpy as jnp
from jax import lax
from jax.experimental import pallas as pl
from jax.experimental.pallas import tpu as pltpu
from jax.experimental.pallas import tpu_sc as plsc

VOCAB = 1000000
EMBED = 64
MAX_LEN = 200
BATCH = 4096
PAD_IDX = 0
EPS = 1e-8

NUM_CORES = 2
NUM_SUBCORES = 16
LANES = 16
NW = NUM_CORES * NUM_SUBCORES
B_PER_W = BATCH // NW
BGROUPS = B_PER_W // LANES
LCHUNK = 20
NLC = MAX_LEN // LCHUNK
NCHUNKS = BGROUPS * NLC
ROWS = LANES * LCHUNK
INV_EMBED = 1.0 / EMBED
NVEC = EMBED // LANES
WIDE = 2 * EMBED


def _rsqrt(z):
    i = plsc.bitcast(z, jnp.int32)
    y = plsc.bitcast(jnp.int32(0x5F3759DF) - (i >> 1), jnp.float32)
    for _ in range(2):
        y = y * (1.5 - 0.5 * z * y * y)
    return y


def _make_kernel():
    mesh = plsc.VectorSubcoreMesh(core_axis_name="c", subcore_axis_name="s")

    @functools.partial(
        pl.kernel,
        mesh=mesh,
        compiler_params=pltpu.CompilerParams(
            needs_layout_passes=False, use_tc_tiling_on_sc=False
        ),
        out_type=jax.ShapeDtypeStruct((BATCH, MAX_LEN, EMBED), jnp.float32),
        scratch_types=[
            pltpu.VMEM((ROWS,), jnp.int32),
            pltpu.VMEM((ROWS,), jnp.int32),
            pltpu.VMEM((ROWS,), jnp.int32),
            pltpu.VMEM((ROWS,), jnp.int32),
            pltpu.VMEM((ROWS, WIDE), jnp.float32),
            pltpu.VMEM((ROWS, WIDE), jnp.float32),
            pltpu.VMEM((ROWS, EMBED), jnp.float32),
            pltpu.VMEM((ROWS, EMBED), jnp.float32),
            pltpu.VMEM((LCHUNK, EMBED), jnp.float32),
            pltpu.VMEM((LCHUNK, EMBED), jnp.float32),
            pltpu.VMEM((EMBED,), jnp.float32),
            pltpu.VMEM((EMBED,), jnp.float32),
            pltpu.SemaphoreType.DMA,
            pltpu.SemaphoreType.DMA,
            pltpu.SemaphoreType.DMA,
            pltpu.SemaphoreType.DMA,
            pltpu.SemaphoreType.DMA,
            pltpu.SemaphoreType.DMA,
        ],
    )
    def emb_kernel(tok_r, w2, w_pos, gamma, beta, out,
                   tokb0, tokb1, idx0, idx1, rows0, rows1, ob0, ob1,
                   pos0, pos1, gamma_v, beta_v,
                   gsem0, gsem1, psem0, psem1, osem0, osem1):
        wid = lax.axis_index("s") * NUM_CORES + lax.axis_index("c")
        lane = lax.iota(jnp.int32, LANES)
        tok_b = (tokb0, tokb1)
        idx_b = (idx0, idx1)
        rows_b = (rows0, rows1)
        ob_b = (ob0, ob1)
        pos_b = (pos0, pos1)
        gsem_b = (gsem0, gsem1)
        psem_b = (psem0, psem1)
        osem_b = (osem0, osem1)

        pltpu.sync_copy(gamma, gamma_v)
        pltpu.sync_copy(beta, beta_v)
        g4 = [gamma_v[pl.ds(LANES * k, LANES)] for k in range(NVEC)]
        b4 = [beta_v[pl.ds(LANES * k, LANES)] for k in range(NVEC)]

        def tok_off(ci):
            return pl.multiple_of((wid * NCHUNKS + ci) * ROWS, 8)

        def pos_copy(ci, p):
            l0 = (ci % NLC) * LCHUNK
            return pltpu.make_async_copy(
                w_pos.at[pl.ds(l0, LCHUNK), :], pos_b[p], psem_b[p]
            )

        def stage_and_gather(ci, p):
            pltpu.sync_copy(tok_r.at[pl.ds(tok_off(ci), ROWS)], tok_b[p])

            def shift(j, carry):
                o = pl.multiple_of(j * LANES, LANES)
                idx_b[p][pl.ds(o, LANES)] = tok_b[p][pl.ds(o, LANES)] >> 1
                return carry
            lax.fori_loop(0, ROWS // LANES, shift, 0)
            pltpu.make_async_copy(
                w2.at[idx_b[p]], rows_b[p], gsem_b[p]
            ).start()
            pos_copy(ci, p).start()

        def wait_gather(ci, p):
            pltpu.make_async_copy(
                w2.at[idx_b[p]], rows_b[p], gsem_b[p]
            ).wait()
            pos_copy(ci, p).wait()

        def out_copy(ci, p, i):
            bg = ci // NLC
            lc = ci % NLC
            b0 = wid * B_PER_W + bg * LANES
            l0 = lc * LCHUNK
            return pltpu.make_async_copy(
                ob_b[p].at[pl.ds(pl.multiple_of(i * LCHUNK, 4), LCHUNK), :],
                out.at[b0 + i, pl.ds(l0, LCHUNK), :],
                osem_b[p],
            )

        def compute(ci, p):
            rows_v = rows_b[p]
            tok_v = tok_b[p]
            pos_v = pos_b[p]
            ob_v = ob_b[p]

            def l_body(l, carry):
                pos4 = [pos_v[l, pl.ds(LANES * k, LANES)]
                        for k in range(NVEC)]
                tokv = plsc.load_gather(tok_v, [lane * LCHUNK + l])
                maskv = jnp.where(tokv != PAD_IDX, 1.0, 0.0)
                halfv = (tokv & 1) * EMBED

                for i in range(LANES):
                    r = i * LCHUNK + l
                    h = halfv[i]
                    x = [
                        rows_v[
                            r,
                            pl.ds(pl.multiple_of(h + LANES * k, LANES), LANES),
                        ]
                        + pos4[k]
                        for k in range(NVEC)
                    ]
                    s = (x[0] + x[1]) + (x[2] + x[3])
                    q = (x[0] * x[0] + x[1] * x[1]) + (x[2] * x[2] + x[3] * x[3])
                    mean = jnp.sum(s) * INV_EMBED
                    var = jnp.sum(q) * INV_EMBED - mean * mean
                    z = jnp.full((LANES,), var + EPS, dtype=jnp.float32)
                    rstd = _rsqrt(z)
                    mf = maskv[i]
                    a = rstd * mf
                    for k in range(NVEC):
                        y = (x[k] - mean) * a * g4[k] + b4[k] * mf
                        ob_v[r, pl.ds(LANES * k, LANES)] = y
                return carry
            lax.fori_loop(0, LCHUNK, l_body, 0)

        stage_and_gather(0, 0)

        def pair_body(h, carry):
            for p in (0, 1):
                ci = h * 2 + p
                wait_gather(ci, p)

                @pl.when(ci + 1 < NCHUNKS)
                def _prep():
                    stage_and_gather(ci + 1, 1 - p)

                @pl.when(ci > 1)
                def _drain():
                    for i in range(LANES):
                        out_copy(ci - 2, p, i).wait()

                compute(ci, p)
                for i in range(LANES):
                    out_copy(ci, p, i).start()
            return carry

        lax.fori_loop(0, NCHUNKS // 2, pair_body, 0)

        for i in range(LANES):
            out_copy(NCHUNKS - 2, 0, i).wait()
        for i in range(LANES):
            out_copy(NCHUNKS - 1, 1, i).wait()

    return emb_kernel


_EMB_KERNEL = _make_kernel()


def kernel(tokens, W_word, W_pos, ln_gamma, ln_beta):
    tok = tokens.astype(jnp.int32)
    tok_r = (
        tok.reshape(NW, BGROUPS, LANES, NLC, LCHUNK)
        .transpose(0, 1, 3, 2, 4)
        .reshape(-1)
    )
    w2 = W_word.reshape(VOCAB // 2, WIDE)
    return _EMB_KERNEL(tok_r, w2, W_pos, ln_gamma, ln_beta)
